# baseline (speedup 1.0000x reference)
import jax
import jax.numpy as jnp
from jax.experimental import pallas as pl

S, E, C = 4096, 16, 512
BS = 32


def _body(g1_ref, l1_ref, g2_ref, l2_ref, m1_ref, m2_ref, o_ref):
    g1m1 = (g1_ref[...] * m1_ref[...])[:, :, None]
    g2m2 = (g2_ref[...] * m2_ref[...])[:, :, None]
    l1 = jnp.broadcast_to(l1_ref[...], (BS, E, C))
    l2 = jnp.broadcast_to(l2_ref[...], (BS, E, C))
    o_ref[...] = g1m1 * l1 + g2m2 * l2


def kernel(gates1_s, locations1_sc, gates2_s, locations2_sc, mask1_float, mask2_float):
    g1 = gates1_s.reshape(S, 1)
    g2 = gates2_s.reshape(S, 1)
    l1 = locations1_sc.reshape(S, 1, C)
    l2 = locations2_sc.reshape(S, 1, C)
    grid = (S // BS,)
    return pl.pallas_call(
        _body,
        grid=grid,
        in_specs=[
            pl.BlockSpec((BS, 1), lambda i: (i, 0)),
            pl.BlockSpec((BS, 1, C), lambda i: (i, 0, 0)),
            pl.BlockSpec((BS, 1), lambda i: (i, 0)),
            pl.BlockSpec((BS, 1, C), lambda i: (i, 0, 0)),
            pl.BlockSpec((BS, E), lambda i: (i, 0)),
            pl.BlockSpec((BS, E), lambda i: (i, 0)),
        ],
        out_specs=pl.BlockSpec((BS, E, C), lambda i: (i, 0, 0)),
        out_shape=jax.ShapeDtypeStruct((S, E, C), jnp.float32),
    )(g1, l1, g2, l2, mask1_float, mask2_float)


# speedup vs baseline: 1.6330x; 1.6330x over previous
import jax
import jax.numpy as jnp
from jax.experimental import pallas as pl

S, E, C = 4096, 16, 512
BS = 128


def _body(g1_ref, l1_ref, g2_ref, l2_ref, m1_ref, m2_ref, o_ref):
    g1m1 = (g1_ref[...] * m1_ref[...])[:, :, None]
    g2m2 = (g2_ref[...] * m2_ref[...])[:, :, None]
    l1 = jnp.broadcast_to(l1_ref[...], (BS, E, C))
    l2 = jnp.broadcast_to(l2_ref[...], (BS, E, C))
    o_ref[...] = g1m1 * l1 + g2m2 * l2


def kernel(gates1_s, locations1_sc, gates2_s, locations2_sc, mask1_float, mask2_float):
    g1 = gates1_s.reshape(S, 1)
    g2 = gates2_s.reshape(S, 1)
    l1 = locations1_sc.reshape(S, 1, C)
    l2 = locations2_sc.reshape(S, 1, C)
    grid = (S // BS,)
    return pl.pallas_call(
        _body,
        grid=grid,
        in_specs=[
            pl.BlockSpec((BS, 1), lambda i: (i, 0)),
            pl.BlockSpec((BS, 1, C), lambda i: (i, 0, 0)),
            pl.BlockSpec((BS, 1), lambda i: (i, 0)),
            pl.BlockSpec((BS, 1, C), lambda i: (i, 0, 0)),
            pl.BlockSpec((BS, E), lambda i: (i, 0)),
            pl.BlockSpec((BS, E), lambda i: (i, 0)),
        ],
        out_specs=pl.BlockSpec((BS, E, C), lambda i: (i, 0, 0)),
        out_shape=jax.ShapeDtypeStruct((S, E, C), jnp.float32),
    )(g1, l1, g2, l2, mask1_float, mask2_float)


# speedup vs baseline: 1.7799x; 1.0900x over previous
import jax
import jax.numpy as jnp
from jax.experimental import pallas as pl

S, E, C = 4096, 16, 512
BS = 256


def _body(g1_ref, l1_ref, g2_ref, l2_ref, m1_ref, m2_ref, o_ref):
    g1m1 = (g1_ref[...] * m1_ref[...])[:, :, None]
    g2m2 = (g2_ref[...] * m2_ref[...])[:, :, None]
    l1 = jnp.broadcast_to(l1_ref[...], (BS, E, C))
    l2 = jnp.broadcast_to(l2_ref[...], (BS, E, C))
    o_ref[...] = g1m1 * l1 + g2m2 * l2


def kernel(gates1_s, locations1_sc, gates2_s, locations2_sc, mask1_float, mask2_float):
    g1 = gates1_s.reshape(S, 1)
    g2 = gates2_s.reshape(S, 1)
    l1 = locations1_sc.reshape(S, 1, C)
    l2 = locations2_sc.reshape(S, 1, C)
    grid = (S // BS,)
    return pl.pallas_call(
        _body,
        grid=grid,
        in_specs=[
            pl.BlockSpec((BS, 1), lambda i: (i, 0)),
            pl.BlockSpec((BS, 1, C), lambda i: (i, 0, 0)),
            pl.BlockSpec((BS, 1), lambda i: (i, 0)),
            pl.BlockSpec((BS, 1, C), lambda i: (i, 0, 0)),
            pl.BlockSpec((BS, E), lambda i: (i, 0)),
            pl.BlockSpec((BS, E), lambda i: (i, 0)),
        ],
        out_specs=pl.BlockSpec((BS, E, C), lambda i: (i, 0, 0)),
        out_shape=jax.ShapeDtypeStruct((S, E, C), jnp.float32),
    )(g1, l1, g2, l2, mask1_float, mask2_float)


# speedup vs baseline: 2.2986x; 1.2914x over previous
import jax
import jax.numpy as jnp
from jax.experimental import pallas as pl

S, E, C = 4096, 16, 512
BS = 256
CH = 4


def _body(g1_ref, l1_ref, g2_ref, l2_ref, m1_ref, m2_ref, o_ref):
    g1m1 = (g1_ref[...] * m1_ref[...])[:, :, None]
    g2m2 = (g2_ref[...] * m2_ref[...])[:, :, None]
    for b in range(0, BS, CH):
        sl = slice(b, b + CH)
        l1 = l1_ref[sl][:, None, :]
        l2 = l2_ref[sl][:, None, :]
        o_ref[sl] = g1m1[sl] * l1 + g2m2[sl] * l2


def kernel(gates1_s, locations1_sc, gates2_s, locations2_sc, mask1_float, mask2_float):
    g1 = gates1_s.reshape(S, 1)
    g2 = gates2_s.reshape(S, 1)
    grid = (S // BS,)
    return pl.pallas_call(
        _body,
        grid=grid,
        in_specs=[
            pl.BlockSpec((BS, 1), lambda i: (i, 0)),
            pl.BlockSpec((BS, C), lambda i: (i, 0)),
            pl.BlockSpec((BS, 1), lambda i: (i, 0)),
            pl.BlockSpec((BS, C), lambda i: (i, 0)),
            pl.BlockSpec((BS, E), lambda i: (i, 0)),
            pl.BlockSpec((BS, E), lambda i: (i, 0)),
        ],
        out_specs=pl.BlockSpec((BS, E, C), lambda i: (i, 0, 0)),
        out_shape=jax.ShapeDtypeStruct((S, E, C), jnp.float32),
    )(g1, locations1_sc, g2, locations2_sc, mask1_float, mask2_float)


# speedup vs baseline: 2.3286x; 1.0131x over previous
import jax
import jax.numpy as jnp
from jax.experimental import pallas as pl

S, E, C = 4096, 16, 512
BS = 512
CH = 4


def _body(g1_ref, l1_ref, g2_ref, l2_ref, m1_ref, m2_ref, o_ref):
    g1m1 = (g1_ref[...] * m1_ref[...])[:, :, None]
    g2m2 = (g2_ref[...] * m2_ref[...])[:, :, None]
    for b in range(0, BS, CH):
        sl = slice(b, b + CH)
        l1 = l1_ref[sl][:, None, :]
        l2 = l2_ref[sl][:, None, :]
        o_ref[sl] = g1m1[sl] * l1 + g2m2[sl] * l2


def kernel(gates1_s, locations1_sc, gates2_s, locations2_sc, mask1_float, mask2_float):
    g1 = gates1_s.reshape(S, 1)
    g2 = gates2_s.reshape(S, 1)
    grid = (S // BS,)
    return pl.pallas_call(
        _body,
        grid=grid,
        in_specs=[
            pl.BlockSpec((BS, 1), lambda i: (i, 0)),
            pl.BlockSpec((BS, C), lambda i: (i, 0)),
            pl.BlockSpec((BS, 1), lambda i: (i, 0)),
            pl.BlockSpec((BS, C), lambda i: (i, 0)),
            pl.BlockSpec((BS, E), lambda i: (i, 0)),
            pl.BlockSpec((BS, E), lambda i: (i, 0)),
        ],
        out_specs=pl.BlockSpec((BS, E, C), lambda i: (i, 0, 0)),
        out_shape=jax.ShapeDtypeStruct((S, E, C), jnp.float32),
    )(g1, locations1_sc, g2, locations2_sc, mask1_float, mask2_float)
